# baseline (speedup 1.0000x reference)
import functools

import jax
import jax.numpy as jnp
from jax import lax
from jax.experimental import pallas as pl
from jax.experimental.pallas import tpu as pltpu
from jax.experimental.pallas import tpu_sc as plsc

NUM_REL = 16
DIM = 64
N_EDGES = 800000

_info = plsc.get_sparse_core_info()
_NC, _NS = _info.num_cores, _info.num_subcores
_NW = _NC * _NS
_B_PER_W = N_EDGES // _NW
_CHUNK = 1000
_N_STEPS = _B_PER_W // _CHUNK


def _make_sc_kernel():
    mesh = plsc.VectorSubcoreMesh(core_axis_name="c", subcore_axis_name="s")

    @functools.partial(
        pl.kernel,
        mesh=mesh,
        compiler_params=pltpu.CompilerParams(use_tc_tiling_on_sc=False),
        out_type=jax.ShapeDtypeStruct((N_EDGES, DIM), jnp.float32),
        scratch_types=[
            pltpu.VMEM((_CHUNK,), jnp.int32),
            pltpu.VMEM((_CHUNK, DIM), jnp.float32),
            pltpu.SemaphoreType.DMA,
        ],
    )
    def k(idx_hbm, table_hbm, out_hbm, idx_v, rows_v, sem):
        wid = lax.axis_index("s") * _NC + lax.axis_index("c")
        base = wid * _B_PER_W

        def step(i, carry):
            off = base + i * _CHUNK
            pltpu.sync_copy(idx_hbm.at[pl.ds(off, _CHUNK)], idx_v)
            pltpu.async_copy(table_hbm.at[idx_v], rows_v, sem).wait()
            pltpu.sync_copy(rows_v, out_hbm.at[pl.ds(off, _CHUNK)])
            return carry

        lax.fori_loop(0, _N_STEPS, step, 0)

    return k


_sc_kernel = _make_sc_kernel()


def kernel(relation_indices, W):
    idx = relation_indices.astype(jnp.int32)
    return _sc_kernel(idx, W)


# speedup vs baseline: 4.6413x; 4.6413x over previous
import functools

import jax
import jax.numpy as jnp
from jax import lax
from jax.experimental import pallas as pl
from jax.experimental.pallas import tpu as pltpu
from jax.experimental.pallas import tpu_sc as plsc

NUM_REL = 16
DIM = 64
N_EDGES = 800000

_info = plsc.get_sparse_core_info()
_NC, _NS = _info.num_cores, _info.num_subcores
_NW = _NC * _NS
_B_PER_W = N_EDGES // _NW
_CHUNK = 1000
_N_STEPS = _B_PER_W // _CHUNK


def _make_sc_kernel():
    mesh = plsc.VectorSubcoreMesh(core_axis_name="c", subcore_axis_name="s")

    @functools.partial(
        pl.kernel,
        mesh=mesh,
        compiler_params=pltpu.CompilerParams(use_tc_tiling_on_sc=False),
        out_type=jax.ShapeDtypeStruct((N_EDGES, DIM), jnp.float32),
        scratch_types=[
            pltpu.VMEM((2, _CHUNK), jnp.int32),
            pltpu.VMEM((_CHUNK, DIM), jnp.float32),
            pltpu.VMEM((_CHUNK, DIM), jnp.float32),
            pltpu.VMEM_SHARED((NUM_REL, DIM), jnp.float32),
            pltpu.SemaphoreType.DMA,
            pltpu.SemaphoreType.DMA,
            pltpu.SemaphoreType.DMA,
            pltpu.SemaphoreType.DMA,
        ],
    )
    def k(idx_hbm, table_hbm, out_hbm, idx_v, rows0, rows1, table_sh,
          sg0, sg1, sw0, sw1):
        cid = lax.axis_index("c")
        sid = lax.axis_index("s")
        wid = sid * _NC + cid
        base = wid * _B_PER_W
        rows = (rows0, rows1)
        sg = (sg0, sg1)
        sw = (sw0, sw1)

        @pl.when(sid == 0)
        def _():
            pltpu.sync_copy(table_hbm, table_sh)

        plsc.subcore_barrier()

        def stage_and_gather(i, b):
            off = base + i * _CHUNK
            pltpu.sync_copy(idx_hbm.at[pl.ds(off, _CHUNK)], idx_v.at[b])
            pltpu.async_copy(table_sh.at[idx_v.at[b]], rows[b], sg[b])

        stage_and_gather(0, 0)
        for i in range(_N_STEPS):
            b = i % 2
            nb = 1 - b
            pltpu.make_async_copy(table_sh.at[idx_v.at[b]], rows[b], sg[b]).wait()
            if i + 1 < _N_STEPS:
                if i >= 1:
                    off_prev = base + (i - 1) * _CHUNK
                    pltpu.make_async_copy(
                        rows[nb], out_hbm.at[pl.ds(off_prev, _CHUNK)], sw[nb]
                    ).wait()
                stage_and_gather(i + 1, nb)
            off = base + i * _CHUNK
            pltpu.async_copy(rows[b], out_hbm.at[pl.ds(off, _CHUNK)], sw[b])
        last = _N_STEPS - 1
        pltpu.make_async_copy(
            rows[last % 2], out_hbm.at[pl.ds(base + last * _CHUNK, _CHUNK)],
            sw[last % 2],
        ).wait()

    return k


_sc_kernel = _make_sc_kernel()


def kernel(relation_indices, W):
    idx = relation_indices.astype(jnp.int32)
    return _sc_kernel(idx, W)
